# baseline (speedup 1.0000x reference)
import functools

import jax
import jax.numpy as jnp
from jax import lax
from jax.experimental import pallas as pl
from jax.experimental.pallas import tpu as pltpu
from jax.experimental.pallas import tpu_sc as plsc

_OBS = 128
_BITS = 64
_ACT = 8
_N = 6
_SIZE = 1200
_EPS = 1e-6
_BATCH = 4096
_GROUP = _SIZE // _ACT
_NC = 2
_NS = 16
_NW = _NC * _NS
_RB = 16
_GACC = _ACT * 16
_CHUNKS = _SIZE // 16

_BSC = 512
_BTC = _BATCH - _BSC
_BPW = _BSC // _NW
_NBLK = _BPW // _RB
_TB = 256
_SIZEP = 1280


def _sc_body(x_hbm, d1_hbm, v1_hbm, m2_hbm, lo1_hbm, hi1_hbm,
             lo2_hbm, hi2_hbm, sidx_hbm, la_hbm, be_hbm, out_hbm,
             x_v, d1_v, v1_v, m2_v, lo1_v, hi1_v, lo2_v, hi2_v,
             sidx_v, la_v, be_v, b1_v, gacc_v, y_v):
    wid = lax.axis_index("s") * _NC + lax.axis_index("c")
    base = wid * _BPW

    pltpu.sync_copy(x_hbm.at[pl.ds(base * _OBS, _BPW * _OBS)], x_v)
    pltpu.sync_copy(d1_hbm, d1_v)
    pltpu.sync_copy(v1_hbm, v1_v)
    pltpu.sync_copy(m2_hbm, m2_v)
    pltpu.sync_copy(lo1_hbm, lo1_v)
    pltpu.sync_copy(hi1_hbm, hi1_v)
    pltpu.sync_copy(lo2_hbm, lo2_v)
    pltpu.sync_copy(hi2_hbm, hi2_v)
    pltpu.sync_copy(sidx_hbm, sidx_v)
    pltpu.sync_copy(la_hbm, la_v)
    pltpu.sync_copy(be_hbm, be_v)

    iota = lax.iota(jnp.int32, 16)
    zero16 = jnp.zeros((16,), jnp.int32)

    ea16 = jnp.exp(la_v[...])
    be16 = be_v[...]

    def block_body(blk, carry):
        row0 = blk * _RB

        def zero_body(i, c):
            gacc_v[pl.ds(i * 16, 16)] = zero16
            return c

        lax.fori_loop(0, (_RB * _GACC) // 16, zero_body, 0)

        @plsc.parallel_loop(0, _CHUNKS, 1, unroll=2)
        def l1_chunk(c):
            off = c * 16
            idxs = [d1_v[pl.ds(j * _SIZE + off, 16)] for j in range(_N)]
            vals = [v1_v[pl.ds(j * _SIZE + off, 16)] for j in range(_N)]
            wlo = lo1_v[pl.ds(off, 16)]
            whi = hi1_v[pl.ds(off, 16)]
            for r in range(_RB):
                xrow = x_v.at[pl.ds((row0 + r) * _OBS, _OBS)]
                t = [jnp.where(plsc.load_gather(xrow, [idxs[j]]) > vals[j],
                               1 << j, 0) for j in range(_N)]
                addr = (t[0] | t[1]) | (t[2] | t[3]) | (t[4] | t[5])
                word = jnp.where(addr >= 32, whi, wlo)
                bit = lax.shift_right_logical(word, addr & 31) & 1
                b1_v[pl.ds(r * _SIZE + off, 16)] = bit

        @plsc.parallel_loop(0, _CHUNKS, 1, unroll=2)
        def l2_chunk(c):
            off = c * 16
            idxs = [m2_v[pl.ds(j * _SIZE + off, 16)] for j in range(_N)]
            wlo = lo2_v[pl.ds(off, 16)]
            whi = hi2_v[pl.ds(off, 16)]
            si = sidx_v[pl.ds(off, 16)]
            for r in range(_RB):
                brow = b1_v.at[pl.ds(r * _SIZE, _SIZE)]
                t = [plsc.load_gather(brow, [idxs[j]]) << j for j in range(_N)]
                addr = (t[0] | t[1]) | (t[2] | t[3]) | (t[4] | t[5])
                word = jnp.where(addr >= 32, whi, wlo)
                bit = lax.shift_right_logical(word, addr & 31) & 1
                plsc.addupdate_scatter(gacc_v.at[pl.ds(r * _GACC, _GACC)],
                                       [si], bit)

        for r in range(_RB):
            gv = zero16
            for k in range(_ACT):
                acc = gacc_v[pl.ds(r * _GACC + k * 16, 16)]
                gv = jnp.where(iota == k, jnp.sum(acc), gv)
            xn = jnp.clip(gv.astype(jnp.float32) / float(_GROUP),
                          _EPS, 1.0 - _EPS)
            y16 = ea16 * (xn - 0.5) + be16
            plsc.store_scatter(y_v, [(row0 + r) * _ACT + iota], y16,
                               mask=iota < _ACT)
        return carry

    lax.fori_loop(0, _NBLK, block_body, 0)
    pltpu.sync_copy(y_v, out_hbm.at[pl.ds(base * _ACT, _BPW * _ACT)])


def _run_sc(x_flat, d1, v1, m2, lo1, hi1, lo2, hi2, sidx, la16, be16):
    mesh = plsc.VectorSubcoreMesh(core_axis_name="c", subcore_axis_name="s")
    run = pl.kernel(
        _sc_body,
        out_type=jax.ShapeDtypeStruct((_BSC * _ACT,), jnp.float32),
        mesh=mesh,
        compiler_params=pltpu.CompilerParams(needs_layout_passes=False),
        scratch_types=[
            pltpu.VMEM((_BPW * _OBS,), jnp.float32),
            pltpu.VMEM((_N * _SIZE,), jnp.int32),
            pltpu.VMEM((_N * _SIZE,), jnp.float32),
            pltpu.VMEM((_N * _SIZE,), jnp.int32),
            pltpu.VMEM((_SIZE,), jnp.int32),
            pltpu.VMEM((_SIZE,), jnp.int32),
            pltpu.VMEM((_SIZE,), jnp.int32),
            pltpu.VMEM((_SIZE,), jnp.int32),
            pltpu.VMEM((_SIZE,), jnp.int32),
            pltpu.VMEM((16,), jnp.float32),
            pltpu.VMEM((16,), jnp.float32),
            pltpu.VMEM((_RB * _SIZE,), jnp.int32),
            pltpu.VMEM((_RB * _GACC,), jnp.int32),
            pltpu.VMEM((_BPW * _ACT,), jnp.float32),
        ],
    )
    out = run(x_flat, d1, v1, m2, lo1, hi1, lo2, hi2, sidx, la16, be16)
    return out.reshape(_BSC, _ACT)


def _tc_body(x_ref, thr_ref, oh_ref, t1_ref, lo1_ref, hi1_ref, w2_ref,
             lo2_ref, hi2_ref, g_ref, la_ref, be_ref, o_ref):
    c = jnp.sum((x_ref[...][:, None, :] > thr_ref[...][None, :, :]
                 ).astype(jnp.float32), axis=1)
    s = jnp.dot(c, oh_ref[...], preferred_element_type=jnp.float32)
    bits = s > t1_ref[...]
    addr = jnp.zeros((_TB, _SIZEP), jnp.int32)
    for j in range(_N):
        addr = addr | (bits[:, j * _SIZEP:(j + 1) * _SIZEP].astype(jnp.int32)
                       << j)
    word = jnp.where(addr >= 32, hi1_ref[...], lo1_ref[...])
    b1 = (lax.shift_right_logical(word, addr & 31) & 1).astype(jnp.float32)
    addr2 = jnp.dot(b1, w2_ref[...],
                    preferred_element_type=jnp.float32).astype(jnp.int32)
    word2 = jnp.where(addr2 >= 32, hi2_ref[...], lo2_ref[...])
    b2 = (lax.shift_right_logical(word2, addr2 & 31) & 1).astype(jnp.float32)
    g = jnp.dot(b2, g_ref[...], preferred_element_type=jnp.float32)
    xn = jnp.clip(g / float(_GROUP), _EPS, 1.0 - _EPS)
    o_ref[...] = jnp.exp(la_ref[...]) * (xn - 0.5) + be_ref[...]


def _run_tc(x_tc, thr2, oh, t1p, lo1p, hi1p, w2p, lo2p, hi2p, gmat, la8, be8):
    full = lambda shp: pl.BlockSpec(shp, lambda i: (0, 0))
    return pl.pallas_call(
        _tc_body,
        grid=(_BTC // _TB,),
        in_specs=[
            pl.BlockSpec((_TB, _OBS), lambda i: (i, 0)),
            full((_BITS, _OBS)),
            full((_OBS, _N * _SIZEP)),
            full((1, _N * _SIZEP)),
            full((1, _SIZEP)),
            full((1, _SIZEP)),
            full((_SIZEP, _SIZEP)),
            full((1, _SIZEP)),
            full((1, _SIZEP)),
            full((_SIZEP, _ACT)),
            full((1, _ACT)),
            full((1, _ACT)),
        ],
        out_specs=pl.BlockSpec((_TB, _ACT), lambda i: (i, 0)),
        out_shape=jax.ShapeDtypeStruct((_BTC, _ACT), jnp.float32),
    )(x_tc, thr2, oh, t1p, lo1p, hi1p, w2p, lo2p, hi2p, gmat, la8, be8)


def _pack_sign_masks(luts):
    s = (luts >= 0).astype(jnp.uint32)
    sh = jnp.arange(32, dtype=jnp.uint32)
    lo = jnp.sum(s[:, :32] << sh, axis=1, dtype=jnp.uint32)
    hi = jnp.sum(s[:, 32:] << sh, axis=1, dtype=jnp.uint32)
    return (lax.bitcast_convert_type(lo, jnp.int32),
            lax.bitcast_convert_type(hi, jnp.int32))


def _pad_row(v, fill):
    return jnp.pad(v, (0, _SIZEP - _SIZE), constant_values=fill)


@jax.jit
def kernel(x, thresholds, mapping1, luts1, mapping2, luts2, log_alpha, beta):
    thr_flat = thresholds.reshape(-1)
    d1 = (mapping1 // _BITS).astype(jnp.int32)
    m2 = mapping2.astype(jnp.int32)
    lo1, hi1 = _pack_sign_masks(luts1)
    lo2, hi2 = _pack_sign_masks(luts2)
    o = jnp.arange(_SIZE, dtype=jnp.int32)

    v1_sc = thr_flat[mapping1].T.reshape(-1)
    d1_sc = d1.T.reshape(-1)
    m2_sc = m2.T.reshape(-1)
    sidx = (o // _GROUP) * 16 + (o % 16)
    la16 = jnp.tile(log_alpha, 2)
    be16 = jnp.tile(beta, 2)

    d1p = jnp.pad(d1.T, ((0, 0), (0, _SIZEP - _SIZE)),
                  constant_values=-1)
    oh = (jnp.arange(_OBS, dtype=jnp.int32)[:, None, None] == d1p[None]
          ).astype(jnp.float32).reshape(_OBS, _N * _SIZEP)
    t1 = (mapping1 % _BITS).astype(jnp.float32)
    t1p = jnp.pad(t1.T, ((0, 0), (0, _SIZEP - _SIZE)),
                  constant_values=jnp.inf).reshape(1, _N * _SIZEP)
    w2t = jnp.sum((m2[:, :, None] == jnp.arange(_SIZEP)[None, None, :]
                   ).astype(jnp.float32)
                  * (2.0 ** jnp.arange(_N))[None, :, None], axis=1)
    w2p = jnp.pad(w2t, ((0, _SIZEP - _SIZE), (0, 0))).T
    gmat = ((o[:, None] // _GROUP) == jnp.arange(_ACT)[None, :]
            ).astype(jnp.float32)
    gmat = jnp.pad(gmat, ((0, _SIZEP - _SIZE), (0, 0)))
    lo1p = _pad_row(lo1, 0)[None, :]
    hi1p = _pad_row(hi1, 0)[None, :]
    lo2p = _pad_row(lo2, 0)[None, :]
    hi2p = _pad_row(hi2, 0)[None, :]

    y_sc = _run_sc(x[:_BSC].reshape(-1), d1_sc, v1_sc, m2_sc,
                   lo1, hi1, lo2, hi2, sidx, la16, be16)
    y_tc = _run_tc(x[_BSC:], thresholds.T, oh, t1p, lo1p, hi1p, w2p, lo2p,
                   hi2p, gmat, log_alpha[None, :], beta[None, :])
    return jnp.concatenate([y_sc, y_tc], axis=0)


# speedup vs baseline: 1.0617x; 1.0617x over previous
import functools

import jax
import jax.numpy as jnp
from jax import lax
from jax.experimental import pallas as pl
from jax.experimental.pallas import tpu as pltpu
from jax.experimental.pallas import tpu_sc as plsc

_OBS = 128
_BITS = 64
_ACT = 8
_N = 6
_SIZE = 1200
_EPS = 1e-6
_BATCH = 4096
_GROUP = _SIZE // _ACT
_NC = 2
_NS = 16
_NW = _NC * _NS
_RB = 16
_GACC = _ACT * 16
_CHUNKS = _SIZE // 16

_BSC = 1024
_BTC = _BATCH - _BSC
_BPW = _BSC // _NW
_NBLK = _BPW // _RB
_TB = 256
_SIZEP = 1280


def _sc_body(x_hbm, d1_hbm, v1_hbm, m2_hbm, lo1_hbm, hi1_hbm,
             lo2_hbm, hi2_hbm, sidx_hbm, la_hbm, be_hbm, out_hbm,
             x_v, d1_v, v1_v, m2_v, lo1_v, hi1_v, lo2_v, hi2_v,
             sidx_v, la_v, be_v, b1_v, gacc_v, y_v):
    wid = lax.axis_index("s") * _NC + lax.axis_index("c")
    base = wid * _BPW

    pltpu.sync_copy(x_hbm.at[pl.ds(base * _OBS, _BPW * _OBS)], x_v)
    pltpu.sync_copy(d1_hbm, d1_v)
    pltpu.sync_copy(v1_hbm, v1_v)
    pltpu.sync_copy(m2_hbm, m2_v)
    pltpu.sync_copy(lo1_hbm, lo1_v)
    pltpu.sync_copy(hi1_hbm, hi1_v)
    pltpu.sync_copy(lo2_hbm, lo2_v)
    pltpu.sync_copy(hi2_hbm, hi2_v)
    pltpu.sync_copy(sidx_hbm, sidx_v)
    pltpu.sync_copy(la_hbm, la_v)
    pltpu.sync_copy(be_hbm, be_v)

    iota = lax.iota(jnp.int32, 16)
    zero16 = jnp.zeros((16,), jnp.int32)

    ea16 = jnp.exp(la_v[...])
    be16 = be_v[...]

    def block_body(blk, carry):
        row0 = blk * _RB

        def zero_body(i, c):
            gacc_v[pl.ds(i * 16, 16)] = zero16
            return c

        lax.fori_loop(0, (_RB * _GACC) // 16, zero_body, 0)

        @plsc.parallel_loop(0, _CHUNKS, 1, unroll=2)
        def l1_chunk(c):
            off = c * 16
            idxs = [d1_v[pl.ds(j * _SIZE + off, 16)] for j in range(_N)]
            vals = [v1_v[pl.ds(j * _SIZE + off, 16)] for j in range(_N)]
            wlo = lo1_v[pl.ds(off, 16)]
            whi = hi1_v[pl.ds(off, 16)]
            for r in range(_RB):
                xrow = x_v.at[pl.ds((row0 + r) * _OBS, _OBS)]
                t = [jnp.where(plsc.load_gather(xrow, [idxs[j]]) > vals[j],
                               1 << j, 0) for j in range(_N)]
                addr = (t[0] | t[1]) | (t[2] | t[3]) | (t[4] | t[5])
                word = jnp.where(addr >= 32, whi, wlo)
                bit = lax.shift_right_logical(word, addr & 31) & 1
                b1_v[pl.ds(r * _SIZE + off, 16)] = bit

        @plsc.parallel_loop(0, _CHUNKS, 1, unroll=2)
        def l2_chunk(c):
            off = c * 16
            idxs = [m2_v[pl.ds(j * _SIZE + off, 16)] for j in range(_N)]
            wlo = lo2_v[pl.ds(off, 16)]
            whi = hi2_v[pl.ds(off, 16)]
            si = sidx_v[pl.ds(off, 16)]
            for r in range(_RB):
                brow = b1_v.at[pl.ds(r * _SIZE, _SIZE)]
                t = [plsc.load_gather(brow, [idxs[j]]) << j for j in range(_N)]
                addr = (t[0] | t[1]) | (t[2] | t[3]) | (t[4] | t[5])
                word = jnp.where(addr >= 32, whi, wlo)
                bit = lax.shift_right_logical(word, addr & 31) & 1
                plsc.addupdate_scatter(gacc_v.at[pl.ds(r * _GACC, _GACC)],
                                       [si], bit)

        for r in range(_RB):
            gv = zero16
            for k in range(_ACT):
                acc = gacc_v[pl.ds(r * _GACC + k * 16, 16)]
                gv = jnp.where(iota == k, jnp.sum(acc), gv)
            xn = jnp.clip(gv.astype(jnp.float32) / float(_GROUP),
                          _EPS, 1.0 - _EPS)
            y16 = ea16 * (xn - 0.5) + be16
            plsc.store_scatter(y_v, [(row0 + r) * _ACT + iota], y16,
                               mask=iota < _ACT)
        return carry

    lax.fori_loop(0, _NBLK, block_body, 0)
    pltpu.sync_copy(y_v, out_hbm.at[pl.ds(base * _ACT, _BPW * _ACT)])


def _run_sc(x_flat, d1, v1, m2, lo1, hi1, lo2, hi2, sidx, la16, be16):
    mesh = plsc.VectorSubcoreMesh(core_axis_name="c", subcore_axis_name="s")
    run = pl.kernel(
        _sc_body,
        out_type=jax.ShapeDtypeStruct((_BSC * _ACT,), jnp.float32),
        mesh=mesh,
        compiler_params=pltpu.CompilerParams(needs_layout_passes=False),
        scratch_types=[
            pltpu.VMEM((_BPW * _OBS,), jnp.float32),
            pltpu.VMEM((_N * _SIZE,), jnp.int32),
            pltpu.VMEM((_N * _SIZE,), jnp.float32),
            pltpu.VMEM((_N * _SIZE,), jnp.int32),
            pltpu.VMEM((_SIZE,), jnp.int32),
            pltpu.VMEM((_SIZE,), jnp.int32),
            pltpu.VMEM((_SIZE,), jnp.int32),
            pltpu.VMEM((_SIZE,), jnp.int32),
            pltpu.VMEM((_SIZE,), jnp.int32),
            pltpu.VMEM((16,), jnp.float32),
            pltpu.VMEM((16,), jnp.float32),
            pltpu.VMEM((_RB * _SIZE,), jnp.int32),
            pltpu.VMEM((_RB * _GACC,), jnp.int32),
            pltpu.VMEM((_BPW * _ACT,), jnp.float32),
        ],
    )
    out = run(x_flat, d1, v1, m2, lo1, hi1, lo2, hi2, sidx, la16, be16)
    return out.reshape(_BSC, _ACT)


def _tc_body(x_ref, thr_ref, oh_ref, t1_ref, lo1_ref, hi1_ref, w2_ref,
             lo2_ref, hi2_ref, g_ref, la_ref, be_ref, o_ref):
    c = jnp.sum((x_ref[...][:, None, :] > thr_ref[...][None, :, :]
                 ).astype(jnp.float32), axis=1)
    s = jnp.dot(c, oh_ref[...], preferred_element_type=jnp.float32)
    bits = s > t1_ref[...]
    addr = jnp.zeros((_TB, _SIZEP), jnp.int32)
    for j in range(_N):
        addr = addr | (bits[:, j * _SIZEP:(j + 1) * _SIZEP].astype(jnp.int32)
                       << j)
    word = jnp.where(addr >= 32, hi1_ref[...], lo1_ref[...])
    b1 = (lax.shift_right_logical(word, addr & 31) & 1).astype(jnp.float32)
    addr2 = jnp.dot(b1, w2_ref[...],
                    preferred_element_type=jnp.float32).astype(jnp.int32)
    word2 = jnp.where(addr2 >= 32, hi2_ref[...], lo2_ref[...])
    b2 = (lax.shift_right_logical(word2, addr2 & 31) & 1).astype(jnp.float32)
    g = jnp.dot(b2, g_ref[...], preferred_element_type=jnp.float32)
    xn = jnp.clip(g / float(_GROUP), _EPS, 1.0 - _EPS)
    o_ref[...] = jnp.exp(la_ref[...]) * (xn - 0.5) + be_ref[...]


def _run_tc(x_tc, thr2, oh, t1p, lo1p, hi1p, w2p, lo2p, hi2p, gmat, la8, be8):
    full = lambda shp: pl.BlockSpec(shp, lambda i: (0, 0))
    return pl.pallas_call(
        _tc_body,
        grid=(_BTC // _TB,),
        in_specs=[
            pl.BlockSpec((_TB, _OBS), lambda i: (i, 0)),
            full((_BITS, _OBS)),
            full((_OBS, _N * _SIZEP)),
            full((1, _N * _SIZEP)),
            full((1, _SIZEP)),
            full((1, _SIZEP)),
            full((_SIZEP, _SIZEP)),
            full((1, _SIZEP)),
            full((1, _SIZEP)),
            full((_SIZEP, _ACT)),
            full((1, _ACT)),
            full((1, _ACT)),
        ],
        out_specs=pl.BlockSpec((_TB, _ACT), lambda i: (i, 0)),
        out_shape=jax.ShapeDtypeStruct((_BTC, _ACT), jnp.float32),
    )(x_tc, thr2, oh, t1p, lo1p, hi1p, w2p, lo2p, hi2p, gmat, la8, be8)


def _pack_sign_masks(luts):
    s = (luts >= 0).astype(jnp.uint32)
    sh = jnp.arange(32, dtype=jnp.uint32)
    lo = jnp.sum(s[:, :32] << sh, axis=1, dtype=jnp.uint32)
    hi = jnp.sum(s[:, 32:] << sh, axis=1, dtype=jnp.uint32)
    return (lax.bitcast_convert_type(lo, jnp.int32),
            lax.bitcast_convert_type(hi, jnp.int32))


def _pad_row(v, fill):
    return jnp.pad(v, (0, _SIZEP - _SIZE), constant_values=fill)


@jax.jit
def kernel(x, thresholds, mapping1, luts1, mapping2, luts2, log_alpha, beta):
    thr_flat = thresholds.reshape(-1)
    d1 = (mapping1 // _BITS).astype(jnp.int32)
    m2 = mapping2.astype(jnp.int32)
    lo1, hi1 = _pack_sign_masks(luts1)
    lo2, hi2 = _pack_sign_masks(luts2)
    o = jnp.arange(_SIZE, dtype=jnp.int32)

    v1_sc = thr_flat[mapping1].T.reshape(-1)
    d1_sc = d1.T.reshape(-1)
    m2_sc = m2.T.reshape(-1)
    sidx = (o // _GROUP) * 16 + (o % 16)
    la16 = jnp.tile(log_alpha, 2)
    be16 = jnp.tile(beta, 2)

    d1p = jnp.pad(d1.T, ((0, 0), (0, _SIZEP - _SIZE)),
                  constant_values=-1)
    oh = (jnp.arange(_OBS, dtype=jnp.int32)[:, None, None] == d1p[None]
          ).astype(jnp.float32).reshape(_OBS, _N * _SIZEP)
    t1 = (mapping1 % _BITS).astype(jnp.float32)
    t1p = jnp.pad(t1.T, ((0, 0), (0, _SIZEP - _SIZE)),
                  constant_values=jnp.inf).reshape(1, _N * _SIZEP)
    w2t = jnp.sum((m2[:, :, None] == jnp.arange(_SIZEP)[None, None, :]
                   ).astype(jnp.float32)
                  * (2.0 ** jnp.arange(_N))[None, :, None], axis=1)
    w2p = jnp.pad(w2t, ((0, _SIZEP - _SIZE), (0, 0))).T
    gmat = ((o[:, None] // _GROUP) == jnp.arange(_ACT)[None, :]
            ).astype(jnp.float32)
    gmat = jnp.pad(gmat, ((0, _SIZEP - _SIZE), (0, 0)))
    lo1p = _pad_row(lo1, 0)[None, :]
    hi1p = _pad_row(hi1, 0)[None, :]
    lo2p = _pad_row(lo2, 0)[None, :]
    hi2p = _pad_row(hi2, 0)[None, :]

    y_sc = _run_sc(x[:_BSC].reshape(-1), d1_sc, v1_sc, m2_sc,
                   lo1, hi1, lo2, hi2, sidx, la16, be16)
    y_tc = _run_tc(x[_BSC:], thresholds.T, oh, t1p, lo1p, hi1p, w2p, lo2p,
                   hi2p, gmat, log_alpha[None, :], beta[None, :])
    return jnp.concatenate([y_sc, y_tc], axis=0)


# speedup vs baseline: 1.1035x; 1.0393x over previous
import functools

import jax
import jax.numpy as jnp
from jax import lax
from jax.experimental import pallas as pl
from jax.experimental.pallas import tpu as pltpu
from jax.experimental.pallas import tpu_sc as plsc

_OBS = 128
_BITS = 64
_ACT = 8
_N = 6
_SIZE = 1200
_EPS = 1e-6
_BATCH = 4096
_GROUP = _SIZE // _ACT
_NC = 2
_NS = 16
_NW = _NC * _NS
_RB = 16
_GACC = _ACT * 16
_CHUNKS = _SIZE // 16

_BSC = 1024
_BTC = _BATCH - _BSC
_BPW = _BSC // _NW
_NBLK = _BPW // _RB
_TB = 256
_SIZEP = 1280


def _sc_body(x_hbm, d1_hbm, v1_hbm, m2_hbm, lo1_hbm, hi1_hbm,
             lo2_hbm, hi2_hbm, sidx_hbm, la_hbm, be_hbm, out_hbm,
             x_v, d1_v, v1_v, m2_v, lo1_v, hi1_v, lo2_v, hi2_v,
             sidx_v, la_v, be_v, b1_v, gacc_v, y_v):
    wid = lax.axis_index("s") * _NC + lax.axis_index("c")
    base = wid * _BPW

    pltpu.sync_copy(x_hbm.at[pl.ds(base * _OBS, _BPW * _OBS)], x_v)
    pltpu.sync_copy(d1_hbm, d1_v)
    pltpu.sync_copy(v1_hbm, v1_v)
    pltpu.sync_copy(m2_hbm, m2_v)
    pltpu.sync_copy(lo1_hbm, lo1_v)
    pltpu.sync_copy(hi1_hbm, hi1_v)
    pltpu.sync_copy(lo2_hbm, lo2_v)
    pltpu.sync_copy(hi2_hbm, hi2_v)
    pltpu.sync_copy(sidx_hbm, sidx_v)
    pltpu.sync_copy(la_hbm, la_v)
    pltpu.sync_copy(be_hbm, be_v)

    iota = lax.iota(jnp.int32, 16)
    zero16 = jnp.zeros((16,), jnp.int32)

    ea16 = jnp.exp(la_v[...])
    be16 = be_v[...]

    def block_body(blk, carry):
        row0 = blk * _RB

        def zero_body(i, c):
            gacc_v[pl.ds(i * 16, 16)] = zero16
            return c

        lax.fori_loop(0, (_RB * _GACC) // 16, zero_body, 0)

        @plsc.parallel_loop(0, _CHUNKS, 1, unroll=2)
        def l1_chunk(c):
            off = c * 16
            idxs = [d1_v[pl.ds(j * _SIZE + off, 16)] for j in range(_N)]
            vals = [v1_v[pl.ds(j * _SIZE + off, 16)] for j in range(_N)]
            wlo = lo1_v[pl.ds(off, 16)]
            whi = hi1_v[pl.ds(off, 16)]
            for r in range(_RB):
                xrow = x_v.at[pl.ds((row0 + r) * _OBS, _OBS)]
                t = [jnp.where(plsc.load_gather(xrow, [idxs[j]]) > vals[j],
                               1 << j, 0) for j in range(_N)]
                addr = (t[0] | t[1]) | (t[2] | t[3]) | (t[4] | t[5])
                word = jnp.where(addr >= 32, whi, wlo)
                bit = lax.shift_right_logical(word, addr & 31) & 1
                b1_v[pl.ds(r * _SIZE + off, 16)] = bit

        @plsc.parallel_loop(0, _CHUNKS, 1, unroll=2)
        def l2_chunk(c):
            off = c * 16
            idxs = [m2_v[pl.ds(j * _SIZE + off, 16)] for j in range(_N)]
            wlo = lo2_v[pl.ds(off, 16)]
            whi = hi2_v[pl.ds(off, 16)]
            si = sidx_v[pl.ds(off, 16)]
            for r in range(_RB):
                brow = b1_v.at[pl.ds(r * _SIZE, _SIZE)]
                t = [plsc.load_gather(brow, [idxs[j]]) << j for j in range(_N)]
                addr = (t[0] | t[1]) | (t[2] | t[3]) | (t[4] | t[5])
                word = jnp.where(addr >= 32, whi, wlo)
                bit = lax.shift_right_logical(word, addr & 31) & 1
                plsc.addupdate_scatter(gacc_v.at[pl.ds(r * _GACC, _GACC)],
                                       [si], bit)

        for r in range(_RB):
            gv = zero16
            for k in range(_ACT):
                acc = gacc_v[pl.ds(r * _GACC + k * 16, 16)]
                gv = jnp.where(iota == k, jnp.sum(acc), gv)
            xn = jnp.clip(gv.astype(jnp.float32) / float(_GROUP),
                          _EPS, 1.0 - _EPS)
            y16 = ea16 * (xn - 0.5) + be16
            plsc.store_scatter(y_v, [(row0 + r) * _ACT + iota], y16,
                               mask=iota < _ACT)
        return carry

    lax.fori_loop(0, _NBLK, block_body, 0)
    pltpu.sync_copy(y_v, out_hbm.at[pl.ds(base * _ACT, _BPW * _ACT)])


def _run_sc(x_flat, d1, v1, m2, lo1, hi1, lo2, hi2, sidx, la16, be16):
    mesh = plsc.VectorSubcoreMesh(core_axis_name="c", subcore_axis_name="s")
    run = pl.kernel(
        _sc_body,
        out_type=jax.ShapeDtypeStruct((_BSC * _ACT,), jnp.float32),
        mesh=mesh,
        compiler_params=pltpu.CompilerParams(needs_layout_passes=False),
        scratch_types=[
            pltpu.VMEM((_BPW * _OBS,), jnp.float32),
            pltpu.VMEM((_N * _SIZE,), jnp.int32),
            pltpu.VMEM((_N * _SIZE,), jnp.float32),
            pltpu.VMEM((_N * _SIZE,), jnp.int32),
            pltpu.VMEM((_SIZE,), jnp.int32),
            pltpu.VMEM((_SIZE,), jnp.int32),
            pltpu.VMEM((_SIZE,), jnp.int32),
            pltpu.VMEM((_SIZE,), jnp.int32),
            pltpu.VMEM((_SIZE,), jnp.int32),
            pltpu.VMEM((16,), jnp.float32),
            pltpu.VMEM((16,), jnp.float32),
            pltpu.VMEM((_RB * _SIZE,), jnp.int32),
            pltpu.VMEM((_RB * _GACC,), jnp.int32),
            pltpu.VMEM((_BPW * _ACT,), jnp.float32),
        ],
    )
    out = run(x_flat, d1, v1, m2, lo1, hi1, lo2, hi2, sidx, la16, be16)
    return out.reshape(_BSC, _ACT)


def _tc_body(x_ref, thr_ref, oh_ref, t1_ref, lo1_ref, hi1_ref, w2_ref,
             lo2_ref, hi2_ref, g_ref, la_ref, be_ref, o_ref):
    c = jnp.sum((x_ref[...][:, None, :] > thr_ref[...][None, :, :]
                 ).astype(jnp.float32), axis=1)
    s = jnp.dot(c, oh_ref[...], preferred_element_type=jnp.float32)
    bits = s > t1_ref[...]
    addr = jnp.zeros((_TB, _SIZEP), jnp.int32)
    for j in range(_N):
        addr = addr | (bits[:, j * _SIZEP:(j + 1) * _SIZEP].astype(jnp.int32)
                       << j)
    word = jnp.where(addr >= 32, hi1_ref[...], lo1_ref[...])
    b1 = (lax.shift_right_logical(word, addr & 31) & 1).astype(jnp.float32)
    addr2 = jnp.dot(b1, w2_ref[...],
                    preferred_element_type=jnp.float32).astype(jnp.int32)
    word2 = jnp.where(addr2 >= 32, hi2_ref[...], lo2_ref[...])
    b2 = (lax.shift_right_logical(word2, addr2 & 31) & 1).astype(jnp.float32)
    g = jnp.dot(b2, g_ref[...], preferred_element_type=jnp.float32)
    xn = jnp.clip(g / float(_GROUP), _EPS, 1.0 - _EPS)
    o_ref[...] = jnp.exp(la_ref[...]) * (xn - 0.5) + be_ref[...]


def _run_tc(x_tc, thr2, oh, t1p, lo1p, hi1p, w2p, lo2p, hi2p, gmat, la8, be8):
    full = lambda shp: pl.BlockSpec(shp, lambda i: (0, 0))
    off = _BSC // _TB
    return pl.pallas_call(
        _tc_body,
        grid=(_BTC // _TB,),
        in_specs=[
            pl.BlockSpec((_TB, _OBS), lambda i: (i + off, 0)),
            full((_BITS, _OBS)),
            full((_OBS, _N * _SIZEP)),
            full((1, _N * _SIZEP)),
            full((1, _SIZEP)),
            full((1, _SIZEP)),
            full((_SIZEP, _SIZEP)),
            full((1, _SIZEP)),
            full((1, _SIZEP)),
            full((_SIZEP, _ACT)),
            full((1, _ACT)),
            full((1, _ACT)),
        ],
        out_specs=pl.BlockSpec((_TB, _ACT), lambda i: (i + off, 0)),
        out_shape=jax.ShapeDtypeStruct((_BATCH, _ACT), jnp.float32),
    )(x_tc, thr2, oh, t1p, lo1p, hi1p, w2p, lo2p, hi2p, gmat, la8, be8)


def _pack_sign_masks(luts):
    s = (luts >= 0).astype(jnp.uint32)
    sh = jnp.arange(32, dtype=jnp.uint32)
    lo = jnp.sum(s[:, :32] << sh, axis=1, dtype=jnp.uint32)
    hi = jnp.sum(s[:, 32:] << sh, axis=1, dtype=jnp.uint32)
    return (lax.bitcast_convert_type(lo, jnp.int32),
            lax.bitcast_convert_type(hi, jnp.int32))


def _pad_row(v, fill):
    return jnp.pad(v, (0, _SIZEP - _SIZE), constant_values=fill)


@jax.jit
def kernel(x, thresholds, mapping1, luts1, mapping2, luts2, log_alpha, beta):
    thr_flat = thresholds.reshape(-1)
    d1 = (mapping1 // _BITS).astype(jnp.int32)
    m2 = mapping2.astype(jnp.int32)
    lo1, hi1 = _pack_sign_masks(luts1)
    lo2, hi2 = _pack_sign_masks(luts2)
    o = jnp.arange(_SIZE, dtype=jnp.int32)

    v1_sc = thr_flat[mapping1].T.reshape(-1)
    d1_sc = d1.T.reshape(-1)
    m2_sc = m2.T.reshape(-1)
    sidx = (o // _GROUP) * 16 + (o % 16)
    la16 = jnp.tile(log_alpha, 2)
    be16 = jnp.tile(beta, 2)

    d1p = jnp.pad(d1.T, ((0, 0), (0, _SIZEP - _SIZE)),
                  constant_values=-1)
    oh = (jnp.arange(_OBS, dtype=jnp.int32)[:, None, None] == d1p[None]
          ).astype(jnp.float32).reshape(_OBS, _N * _SIZEP)
    t1 = (mapping1 % _BITS).astype(jnp.float32)
    t1p = jnp.pad(t1.T, ((0, 0), (0, _SIZEP - _SIZE)),
                  constant_values=jnp.inf).reshape(1, _N * _SIZEP)
    w2t = jnp.sum((m2[:, :, None] == jnp.arange(_SIZEP)[None, None, :]
                   ).astype(jnp.float32)
                  * (2.0 ** jnp.arange(_N))[None, :, None], axis=1)
    w2p = jnp.pad(w2t, ((0, _SIZEP - _SIZE), (0, 0))).T
    gmat = ((o[:, None] // _GROUP) == jnp.arange(_ACT)[None, :]
            ).astype(jnp.float32)
    gmat = jnp.pad(gmat, ((0, _SIZEP - _SIZE), (0, 0)))
    lo1p = _pad_row(lo1, 0)[None, :]
    hi1p = _pad_row(hi1, 0)[None, :]
    lo2p = _pad_row(lo2, 0)[None, :]
    hi2p = _pad_row(hi2, 0)[None, :]

    y_sc = _run_sc(x.reshape(-1), d1_sc, v1_sc, m2_sc,
                   lo1, hi1, lo2, hi2, sidx, la16, be16)
    y_tc = _run_tc(x, thresholds.T, oh, t1p, lo1p, hi1p, w2p, lo2p,
                   hi2p, gmat, log_alpha[None, :], beta[None, :])
    return lax.dynamic_update_slice(y_tc, y_sc, (0, 0))
